# baseline (speedup 1.0000x reference)
import jax
import jax.numpy as jnp
from jax.experimental import pallas as pl

_BS = 1024


def _add_kernel(x_ref, w_ref, o_ref):
    o_ref[0] = x_ref[0] + w_ref[...]


def kernel(x, weight):
    B, S, H = x.shape
    w = weight[:S]
    grid = (S // _BS, B)
    return pl.pallas_call(
        _add_kernel,
        grid=grid,
        in_specs=[
            pl.BlockSpec((1, _BS, H), lambda i, j: (j, i, 0)),
            pl.BlockSpec((_BS, H), lambda i, j: (i, 0)),
        ],
        out_specs=pl.BlockSpec((1, _BS, H), lambda i, j: (j, i, 0)),
        out_shape=jax.ShapeDtypeStruct(x.shape, x.dtype),
    )(x, w)


# speedup vs baseline: 1.0392x; 1.0392x over previous
import jax
import jax.numpy as jnp
from jax.experimental import pallas as pl

_BS = 2048


def _add_kernel(x_ref, w_ref, o_ref):
    o_ref[...] = x_ref[...] + w_ref[...]


def kernel(x, weight):
    B, S, H = x.shape
    w = weight[:S]
    x2 = x.reshape(B * S, H)
    nsb = S // _BS
    out = pl.pallas_call(
        _add_kernel,
        grid=(nsb, B),
        in_specs=[
            pl.BlockSpec((_BS, H), lambda i, j: (j * nsb + i, 0)),
            pl.BlockSpec((_BS, H), lambda i, j: (i, 0)),
        ],
        out_specs=pl.BlockSpec((_BS, H), lambda i, j: (j * nsb + i, 0)),
        out_shape=jax.ShapeDtypeStruct((B * S, H), x.dtype),
    )(x2, w)
    return out.reshape(B, S, H)
